# baseline (speedup 1.0000x reference)
import jax
import jax.numpy as jnp
from jax import lax
from jax.experimental import pallas as pl
from jax.experimental.pallas import tpu as pltpu
from jax.experimental.pallas import tpu_sc as plsc

VOCAB = 1000
HIDDEN = 128
BATCH = 1024
SEQ = 20
TOKENS = BATCH * SEQ

VPAD = 1024
CROWS = 1024

NC, NS, L = 2, 16, 16
NW = NC * NS
RPW = TOKENS // NW
R = 16
CHNK = R * VOCAB
NCHUNK = RPW // R
NBUF = 2


def _build_table_body(ew_ref, wt_ref, c_ref):
    c_ref[...] = jnp.dot(ew_ref[...], wt_ref[...],
                         preferred_element_type=jnp.float32)


def _build_table(embed_tokens, embed_positions, lm_head_w):
    ew = jnp.concatenate(
        [embed_tokens, embed_positions[:SEQ],
         jnp.zeros((CROWS - VOCAB - SEQ, HIDDEN), jnp.float32)], axis=0)
    wt = jnp.pad(lm_head_w, ((0, VPAD - VOCAB), (0, 0))).T
    return pl.pallas_call(
        _build_table_body,
        out_shape=jax.ShapeDtypeStruct((CROWS, VPAD), jnp.float32),
    )(ew, wt)


def _sc_body(c_hbm, ids_hbm, out_hbm, idx_v, p2_v, buf0, buf1, ob0, ob1,
             gsem0, gsem1, osem0, osem1):
    bufs = (buf0, buf1)
    obufs = (ob0, ob1)
    gsems = (gsem0, gsem1)
    osems = (osem0, osem1)
    wid = lax.axis_index("s") * NC + lax.axis_index("c")
    tok0 = wid * RPW

    pltpu.sync_copy(ids_hbm.at[pl.ds(tok0, RPW)], idx_v)
    pltpu.sync_copy(c_hbm.at[pl.ds(VOCAB, p2_v.shape[0]), :], p2_v)

    def gather_copy(g, b):
        return pltpu.make_async_copy(
            c_hbm.at[idx_v.at[pl.ds(g * R, R)]], bufs[b], gsems[b])

    def drain_copy(g, b):
        dst = out_hbm.at[pl.ds((tok0 + g * R) * VOCAB, CHNK)]
        return pltpu.make_async_copy(obufs[b].at[pl.ds(0, CHNK)], dst,
                                     osems[b])

    for b in range(NBUF):
        gather_copy(b, b).start()

    def add_row(r, g, b):
        s = lax.rem(g * R + r, SEQ)
        base = r * VOCAB
        for j in range(63):
            sl = pl.ds(j * L, L)
            obufs[b][pl.ds(base + j * L, L)] = bufs[b][r, sl] + p2_v[s, sl]
        return r

    def outer(i, carry):
        gi = i * NBUF
        for b in range(NBUF):
            g = gi + b
            gather_copy(g, b).wait()

            @pl.when(g >= NBUF)
            def _():
                drain_copy(g - NBUF, b).wait()

            lax.fori_loop(0, R, lambda r, _: add_row(r, g, b), 0)
            drain_copy(g, b).start()

            @pl.when(g + NBUF < NCHUNK)
            def _():
                gather_copy(g + NBUF, b).start()
        return carry

    lax.fori_loop(0, NCHUNK // NBUF, outer, 0)

    for b in range(NBUF):
        drain_copy(NCHUNK - NBUF + b, b).wait()


def _sc_gather(c, ids_flat):
    mesh = plsc.VectorSubcoreMesh(core_axis_name="c", subcore_axis_name="s")
    f = pl.kernel(
        _sc_body,
        mesh=mesh,
        out_type=jax.ShapeDtypeStruct((TOKENS * VOCAB,), jnp.float32),
        scratch_types=[
            pltpu.VMEM((RPW,), jnp.int32),
            pltpu.VMEM((24, VPAD), jnp.float32),
            pltpu.VMEM((R, VPAD), jnp.float32),
            pltpu.VMEM((R, VPAD), jnp.float32),
            pltpu.VMEM((CHNK + 8, ), jnp.float32),
            pltpu.VMEM((CHNK + 8, ), jnp.float32),
            pltpu.SemaphoreType.DMA,
            pltpu.SemaphoreType.DMA,
            pltpu.SemaphoreType.DMA,
            pltpu.SemaphoreType.DMA,
        ],
    )
    return f(c, ids_flat)


def kernel(input_ids, embed_tokens, embed_positions, lm_head_w):
    c = _build_table(embed_tokens, embed_positions, lm_head_w)
    out = _sc_gather(c, input_ids.reshape(TOKENS))
    return out.reshape(BATCH, SEQ, VOCAB)


# speedup vs baseline: 2.0332x; 2.0332x over previous
import jax
import jax.numpy as jnp
from jax import lax
from jax.experimental import pallas as pl
from jax.experimental.pallas import tpu as pltpu
from jax.experimental.pallas import tpu_sc as plsc

VOCAB = 1000
HIDDEN = 128
BATCH = 1024
SEQ = 20
TOKENS = BATCH * SEQ

VPAD = 1024
CROWS = 1024

NC, NS, L = 2, 16, 16
NW = NC * NS
RPW = TOKENS // NW
G = 16
BODYROWS = 160
NBODY = RPW // BODYROWS
SLOTS = BODYROWS // G
BATCHES = BODYROWS // SEQ
NSLOT = RPW // G
NJ = VPAD // L - 1
DEPTH = 3


def _build_table_body(ew_ref, wt_ref, c_ref):
    c_ref[...] = jnp.dot(ew_ref[...], wt_ref[...],
                         preferred_element_type=jnp.float32)


def _build_table(embed_tokens, embed_positions, lm_head_w):
    ew = jnp.concatenate(
        [embed_tokens, embed_positions[:SEQ],
         jnp.zeros((CROWS - VOCAB - SEQ, HIDDEN), jnp.float32)], axis=0)
    wt = jnp.pad(lm_head_w, ((0, VPAD - VOCAB), (0, 0))).T
    return pl.pallas_call(
        _build_table_body,
        out_shape=jax.ShapeDtypeStruct((CROWS, VPAD), jnp.float32),
    )(ew, wt)


def _lane_off(j):
    return min(j * L, VOCAB - L)


def _segments(k):
    lo, hi = k * G, (k + 1) * G
    segs = []
    while lo < hi:
        nxt = min(hi, (lo // SEQ + 1) * SEQ)
        segs.append((lo // SEQ, lo % SEQ, lo - k * G, nxt - lo))
        lo = nxt
    return segs


def _sc_body(c_hbm, ids_hbm, out_hbm, idx_v, p2_v, buf0, buf1, ob0, ob1,
             gsem0, gsem1, osem0, osem1):
    bufs = (buf0, buf1)
    obufs = (ob0, ob1)
    gsems = (gsem0, gsem1)
    osems = (osem0, osem1)
    cid = lax.axis_index("c")
    sid = lax.axis_index("s")
    wid = sid * NC + cid
    tok0 = wid * RPW
    bat0 = wid * (RPW // SEQ)

    pltpu.sync_copy(ids_hbm.at[pl.ds(tok0, RPW)], idx_v)
    pltpu.sync_copy(c_hbm.at[pl.ds(VOCAB, p2_v.shape[0]), :], p2_v)

    def start_gather(slot, b):
        pltpu.make_async_copy(c_hbm.at[idx_v.at[pl.ds(slot * G, G)]],
                              bufs[b], gsems[b]).start()

    def wait_gather(b):
        pltpu.make_async_copy(c_hbm.at[pl.ds(0, G), :], bufs[b],
                              gsems[b]).wait()

    def start_drain(mg, m):
        pltpu.make_async_copy(obufs[m % 2], out_hbm.at[bat0 + mg],
                              osems[m % 2]).start()

    def wait_drain(m):
        pltpu.make_async_copy(obufs[m % 2], out_hbm.at[0],
                              osems[m % 2]).wait()

    for k in range(2):
        start_gather(k, k)

    def add_rows(b, m, pos0, br0, ln):
        def add_row(r, carry):
            acc = {}
            for j in range(NJ + DEPTH):
                if j < NJ:
                    sl = pl.ds(_lane_off(j), L)
                    acc[j] = (bufs[b][br0 + r, sl], p2_v[pos0 + r, sl])
                if j >= DEPTH:
                    k2 = j - DEPTH
                    a, p = acc.pop(k2)
                    obufs[m % 2][pos0 + r, pl.ds(_lane_off(k2), L)] = a + p
            return carry

        lax.fori_loop(0, ln, add_row, 0)

    def body(q, carry):
        slot0 = q * SLOTS
        mg0 = q * BATCHES
        for k in range(SLOTS):
            wait_gather(k % 2)
            for (m, pos0, br0, ln) in _segments(k):
                if pos0 == 0:
                    @pl.when(mg0 + m >= 2)
                    def _():
                        wait_drain(m)
                add_rows(k % 2, m, pos0, br0, ln)
                if pos0 + ln == SEQ:
                    start_drain(mg0 + m, m)

            @pl.when(slot0 + k + 2 < NSLOT)
            def _():
                start_gather(slot0 + k + 2, k % 2)
        return carry

    lax.fori_loop(0, NBODY, body, 0)

    for m in range(2):
        wait_drain(m)


def _sc_gather(c, ids_flat):
    mesh = plsc.VectorSubcoreMesh(core_axis_name="c", subcore_axis_name="s")
    f = pl.kernel(
        _sc_body,
        mesh=mesh,
        out_type=jax.ShapeDtypeStruct((BATCH, SEQ, VOCAB), jnp.float32),
        scratch_types=[
            pltpu.VMEM((RPW,), jnp.int32),
            pltpu.VMEM((24, VPAD), jnp.float32),
            pltpu.VMEM((G, VPAD), jnp.float32),
            pltpu.VMEM((G, VPAD), jnp.float32),
            pltpu.VMEM((SEQ, VOCAB), jnp.float32),
            pltpu.VMEM((SEQ, VOCAB), jnp.float32),
        ] + [pltpu.SemaphoreType.DMA] * 4,
    )
    return f(c, ids_flat)


def kernel(input_ids, embed_tokens, embed_positions, lm_head_w):
    c = _build_table(embed_tokens, embed_positions, lm_head_w)
    return _sc_gather(c, input_ids.reshape(TOKENS))
